# baseline (speedup 1.0000x reference)
import functools

import jax
import jax.numpy as jnp
from jax import lax
from jax.experimental import pallas as pl
from jax.experimental.pallas import tpu as pltpu
from jax.experimental.pallas import tpu_sc as plsc

MAXLEN = 200
EMB = 64
NUM_TILES = 32


def _tok_pos_embed(x_flat, token_table, pos_table):
    total = x_flat.shape[0]
    rows_per_tile = total // NUM_TILES
    seqs_per_tile = rows_per_tile // MAXLEN
    mesh = plsc.VectorSubcoreMesh(core_axis_name="c", subcore_axis_name="s")

    @functools.partial(
        pl.kernel,
        out_type=jax.ShapeDtypeStruct((total, EMB), jnp.float32),
        mesh=mesh,
        compiler_params=pltpu.CompilerParams(use_tc_tiling_on_sc=False),
        scratch_types=[
            pltpu.VMEM((rows_per_tile,), jnp.int32),
            pltpu.VMEM((MAXLEN, EMB), jnp.float32),
            pltpu.VMEM((MAXLEN, EMB), jnp.float32),
            pltpu.SemaphoreType.DMA,
        ],
    )
    def k(x_hbm, tok_hbm, pos_hbm, out_hbm, idx_v, pos_v, rows_v, sem):
        wid = lax.axis_index("s") * 2 + lax.axis_index("c")
        base = wid * rows_per_tile
        pltpu.sync_copy(x_hbm.at[pl.ds(base, rows_per_tile)], idx_v)
        pltpu.sync_copy(pos_hbm, pos_v)

        @pl.loop(0, seqs_per_tile)
        def _seq(s):
            off = s * MAXLEN
            cp0 = pltpu.async_copy(
                tok_hbm.at[idx_v.at[pl.ds(off, 128)]],
                rows_v.at[pl.ds(0, 128)], sem)
            cp1 = pltpu.async_copy(
                tok_hbm.at[idx_v.at[pl.ds(off + 128, 72)]],
                rows_v.at[pl.ds(128, 72)], sem)
            cp0.wait()
            cp1.wait()

            @pl.loop(0, MAXLEN)
            def _row(r):
                for c in range(0, EMB, 16):
                    plsc.addupdate(rows_v.at[r, pl.ds(c, 16)],
                                   pos_v[r, pl.ds(c, 16)])

            pltpu.sync_copy(rows_v, out_hbm.at[pl.ds(base + off, MAXLEN)])

    return k(x_flat, token_table, pos_table)


def kernel(x, token_table, pos_table):
    batch, seq = x.shape
    if seq < MAXLEN:
        x = jnp.pad(x, ((0, 0), (0, MAXLEN - seq)))
    else:
        x = x[:, :MAXLEN]
    x_flat = x.reshape(-1).astype(jnp.int32)
    out = _tok_pos_embed(x_flat, token_table, pos_table)
    return out.reshape(batch, MAXLEN, EMB)


# speedup vs baseline: 1.1637x; 1.1637x over previous
import functools

import jax
import jax.numpy as jnp
from jax import lax
from jax.experimental import pallas as pl
from jax.experimental.pallas import tpu as pltpu
from jax.experimental.pallas import tpu_sc as plsc

MAXLEN = 200
EMB = 64
NUM_TILES = 32
NBUF = 4
SPLIT = (104, 96)


def _tok_pos_embed(x_flat, token_table, pos_table):
    total = x_flat.shape[0]
    rows_per_tile = total // NUM_TILES
    nchunk = rows_per_tile // MAXLEN
    mesh = plsc.VectorSubcoreMesh(core_axis_name="c", subcore_axis_name="s")

    @functools.partial(
        pl.kernel,
        out_type=jax.ShapeDtypeStruct((total, EMB), jnp.float32),
        mesh=mesh,
        compiler_params=pltpu.CompilerParams(use_tc_tiling_on_sc=False),
        scratch_types=[
            pltpu.VMEM((rows_per_tile,), jnp.int32),
            pltpu.VMEM((MAXLEN, EMB), jnp.float32),
        ] + [pltpu.VMEM((MAXLEN, EMB), jnp.float32) for _ in range(NBUF)]
          + [pltpu.SemaphoreType.DMA for _ in range(2 * NBUF)],
    )
    def k(x_hbm, tok_hbm, pos_hbm, out_hbm, idx_v, pos_v, *bufs_and_sems):
        bufs = bufs_and_sems[:NBUF]
        gsems = bufs_and_sems[NBUF:2 * NBUF]
        osems = bufs_and_sems[2 * NBUF:]
        wid = lax.axis_index("s") * 2 + lax.axis_index("c")
        base = wid * rows_per_tile
        pltpu.sync_copy(x_hbm.at[pl.ds(base, rows_per_tile)], idx_v)
        pltpu.sync_copy(pos_hbm, pos_v)

        def issue_gather(c, b):
            off = c * MAXLEN
            r0 = 0
            for n in SPLIT:
                pltpu.async_copy(
                    tok_hbm.at[idx_v.at[pl.ds(off + r0, n)]],
                    bufs[b].at[pl.ds(r0, n)], gsems[b])
                r0 += n

        def wait_gather(c, b):
            off = c * MAXLEN
            r0 = 0
            for n in SPLIT:
                pltpu.make_async_copy(
                    tok_hbm.at[idx_v.at[pl.ds(off + r0, n)]],
                    bufs[b].at[pl.ds(r0, n)], gsems[b]).wait()
                r0 += n

        def issue_out(c, b):
            pltpu.async_copy(bufs[b], out_hbm.at[pl.ds(base + c * MAXLEN, MAXLEN)],
                             osems[b])

        def wait_out(c, b):
            pltpu.make_async_copy(bufs[b],
                                  out_hbm.at[pl.ds(base + c * MAXLEN, MAXLEN)],
                                  osems[b]).wait()

        issue_gather(0, 0)
        issue_gather(1, 1)

        @pl.loop(0, nchunk, step=NBUF)
        def _grp(g):
            for b in range(NBUF):
                c = g + b
                bp = (b + 2) % NBUF
                wait_gather(c, b)

                @pl.when(c + 2 < nchunk)
                def _prefetch():
                    @pl.when(c >= 2)
                    def _drain():
                        wait_out(c - 2, bp)
                    issue_gather(c + 2, bp)

                @pl.loop(0, MAXLEN, unroll=4)
                def _row(r):
                    for col in range(0, EMB, 16):
                        plsc.addupdate(bufs[b].at[r, pl.ds(col, 16)],
                                       pos_v[r, pl.ds(col, 16)])

                issue_out(c, b)

        for b in range(NBUF):
            wait_out(nchunk - NBUF + b, b)

    return k(x_flat, token_table, pos_table)


def kernel(x, token_table, pos_table):
    batch, seq = x.shape
    if seq < MAXLEN:
        x = jnp.pad(x, ((0, 0), (0, MAXLEN - seq)))
    else:
        x = x[:, :MAXLEN]
    x_flat = x.reshape(-1).astype(jnp.int32)
    out = _tok_pos_embed(x_flat, token_table, pos_table)
    return out.reshape(batch, MAXLEN, EMB)
